# Initial kernel scaffold; baseline (speedup 1.0000x reference)
import jax
import jax.numpy as jnp
from jax.experimental import pallas as pl


def kernel(X, field, convert):
    raise NotImplementedError("write your pallas kernel here")


# speedup vs baseline: 6.2755x; 6.2755x over previous
import functools

import jax
import jax.numpy as jnp
from jax import lax
from jax.experimental import pallas as pl
from jax.experimental.pallas import tpu as pltpu
from jax.experimental.pallas import tpu_sc as plsc

N_B = 2
C_CH = 128
IN = 24
NSRC = IN * IN
OUT = 12
NCELL = OUT * OUT
ROWW = 272
NCORES = 2
NSUB = 16
PER_TILE = 24
ACTIVE = NSRC // PER_TILE
ACC_CHUNK = 16
ACC_CHUNKS = NCELL // ACC_CHUNK


def _prep_body(x_ref, f_ref, cv_ref, rows_ref, dest_ref, aux_ref):
    X = x_ref[...]
    r = jnp.sqrt(jnp.sum(X * X, axis=1))
    M = jnp.maximum(jnp.max(r, axis=1, keepdims=True), 0.0)
    e = jnp.exp(r - M)
    d = f_ref[...] * (1.0 - jnp.tanh(r))
    z = jnp.round((d + 1.0) / cv_ref[...]).astype(jnp.int32)
    dest_ref[...] = z[0:1] * OUT + z[1:2]
    A = (e[:, None, :] * X).reshape(N_B * C_CH, NSRC)
    rows_ref[...] = jnp.concatenate(
        [
            A.T,
            e.T,
            jnp.ones((NSRC, 1), jnp.float32),
            jnp.zeros((NSRC, ROWW - (N_B * C_CH + N_B + 1)), jnp.float32),
        ],
        axis=1,
    )
    aux_ref[...] = jnp.exp(-M).T


def _prep_call(Xr, fieldr, convr):
    return pl.pallas_call(
        _prep_body,
        out_shape=(
            jax.ShapeDtypeStruct((NSRC, ROWW), jnp.float32),
            jax.ShapeDtypeStruct((1, NSRC), jnp.int32),
            jax.ShapeDtypeStruct((1, N_B), jnp.float32),
        ),
    )(Xr, fieldr, convr)


def _scatter_body(rows_hbm, dest_hbm, out_hbm, idx_v, rows_v, buf_v, acc_sh):
    c = lax.axis_index("c")
    s = lax.axis_index("s")
    wid = s * NCORES + c

    zero16 = jnp.zeros((16,), jnp.float32)
    for k in range(ACC_CHUNK):
        for t in range(ROWW // 16):
            buf_v[k, pl.ds(t * 16, 16)] = zero16

    @pl.when(s < ACC_CHUNKS)
    def _zero():
        pltpu.sync_copy(buf_v, acc_sh.at[pl.ds(s * ACC_CHUNK, ACC_CHUNK)])

    plsc.subcore_barrier()

    @pl.when(wid < ACTIVE)
    def _scatter():
        pltpu.sync_copy(dest_hbm.at[pl.ds(wid * PER_TILE, PER_TILE)], idx_v)
        pltpu.sync_copy(rows_hbm.at[pl.ds(wid * PER_TILE, PER_TILE)], rows_v)
        pltpu.sync_copy(rows_v, acc_sh.at[idx_v], add=True)

    plsc.subcore_barrier()

    @pl.when(s < ACC_CHUNKS)
    def _writeout():
        pltpu.sync_copy(acc_sh.at[pl.ds(s * ACC_CHUNK, ACC_CHUNK)], buf_v)
        pltpu.sync_copy(buf_v, out_hbm.at[c, pl.ds(s * ACC_CHUNK, ACC_CHUNK)])


@functools.cache
def _scatter_call():
    mesh = plsc.VectorSubcoreMesh(
        core_axis_name="c", subcore_axis_name="s",
        num_cores=NCORES, num_subcores=NSUB,
    )
    return pl.kernel(
        _scatter_body,
        mesh=mesh,
        compiler_params=pltpu.CompilerParams(use_tc_tiling_on_sc=False),
        out_type=jax.ShapeDtypeStruct((NCORES, NCELL, ROWW), jnp.float32),
        scratch_types=[
            pltpu.VMEM((PER_TILE,), jnp.int32),
            pltpu.VMEM((PER_TILE, ROWW), jnp.float32),
            pltpu.VMEM((ACC_CHUNK, ROWW), jnp.float32),
            pltpu.VMEM_SHARED((NCELL, ROWW), jnp.float32),
        ],
    )


def _finish_body(parts_ref, aux_ref, o_ref):
    acc = parts_ref[0] + parts_ref[1]
    base = float(NSRC) - acc[:, N_B * C_CH + N_B]
    for n in range(N_B):
        denom = acc[:, N_B * C_CH + n] + base * aux_ref[0, n]
        numer = acc[:, n * C_CH:(n + 1) * C_CH]
        o_ref[n] = (numer / denom[:, None]).T


def _finish_call(parts, aux):
    return pl.pallas_call(
        _finish_body,
        out_shape=jax.ShapeDtypeStruct((N_B, C_CH, NCELL), jnp.float32),
    )(parts, aux)


def kernel(X, field, convert):
    Xr = X.reshape(N_B, C_CH, NSRC)
    fieldr = field.reshape(N_B, NSRC)
    convr = jnp.broadcast_to(convert.reshape(N_B, 1), (N_B, NSRC))
    rows, dest, aux = _prep_call(Xr, fieldr, convr)
    parts = _scatter_call()(rows, dest.reshape(NSRC))
    out = _finish_call(parts, aux)
    return out.reshape(N_B, C_CH, OUT, OUT)


# speedup vs baseline: 6.4159x; 1.0224x over previous
import functools

import jax
import jax.numpy as jnp
from jax import lax
from jax.experimental import pallas as pl
from jax.experimental.pallas import tpu as pltpu
from jax.experimental.pallas import tpu_sc as plsc

N_B = 2
C_CH = 128
IN = 24
NSRC = IN * IN
OUT = 12
NCELL = OUT * OUT
NBAND = 3
ACC_R = NBAND * NCELL
NCORES = 2
NSUB = 16
PER_TILE = 24
ACTIVE = NSRC // PER_TILE
ACC_PER_SUB = ACC_R // NSUB


def _prep_body(x_ref, f_ref, cv_ref, r0_ref, r1_ref, st_ref, d3_ref, aux_ref):
    X = x_ref[...].reshape(N_B, C_CH, NSRC)
    F = f_ref[...].reshape(N_B, NSRC)
    r = jnp.sqrt(jnp.sum(X * X, axis=1))
    M = jnp.maximum(jnp.max(r, axis=1, keepdims=True), 0.0)
    e = jnp.exp(r - M)
    d = F * (1.0 - jnp.tanh(r))
    z = jnp.round((d + 1.0) / cv_ref[...]).astype(jnp.int32)
    dest = z[0:1] * OUT + z[1:2]
    d3_ref[...] = jnp.concatenate(
        [dest, dest + NCELL, dest + 2 * NCELL], axis=1)
    r0_ref[...] = (e[0:1] * X[0]).T
    r1_ref[...] = (e[1:2] * X[1]).T
    st_ref[...] = jnp.concatenate(
        [
            e.T,
            jnp.ones((NSRC, 1), jnp.float32),
            jnp.zeros((NSRC, C_CH - N_B - 1), jnp.float32),
        ],
        axis=1,
    )
    aux_ref[...] = jnp.exp(-M)


def _prep_call(X, field, cv):
    return pl.pallas_call(
        _prep_body,
        out_shape=(
            jax.ShapeDtypeStruct((NSRC, C_CH), jnp.float32),
            jax.ShapeDtypeStruct((NSRC, C_CH), jnp.float32),
            jax.ShapeDtypeStruct((NSRC, C_CH), jnp.float32),
            jax.ShapeDtypeStruct((1, NBAND * NSRC), jnp.int32),
            jax.ShapeDtypeStruct((N_B, 1), jnp.float32),
        ),
    )(X, field, cv)


def _scatter_body(r0_hbm, r1_hbm, st_hbm, d3_hbm, out_hbm,
                  i0_v, i1_v, i2_v, b0_v, b1_v, b2_v, w_v, acc_sh):
    c = lax.axis_index("c")
    s = lax.axis_index("s")
    wid = s * NCORES + c

    zero16 = jnp.zeros((16,), jnp.float32)
    for k in range(ACC_PER_SUB):
        for t in range(C_CH // 16):
            w_v[k, pl.ds(t * 16, 16)] = zero16
    pltpu.sync_copy(w_v, acc_sh.at[pl.ds(s * ACC_PER_SUB, ACC_PER_SUB)])
    plsc.subcore_barrier()

    @pl.when(wid < ACTIVE)
    def _scatter():
        base = wid * PER_TILE
        pltpu.sync_copy(d3_hbm.at[pl.ds(base, PER_TILE)], i0_v)
        pltpu.sync_copy(d3_hbm.at[pl.ds(NSRC + base, PER_TILE)], i1_v)
        pltpu.sync_copy(d3_hbm.at[pl.ds(2 * NSRC + base, PER_TILE)], i2_v)
        pltpu.sync_copy(r0_hbm.at[pl.ds(base, PER_TILE)], b0_v)
        pltpu.sync_copy(r1_hbm.at[pl.ds(base, PER_TILE)], b1_v)
        pltpu.sync_copy(st_hbm.at[pl.ds(base, PER_TILE)], b2_v)
        pltpu.sync_copy(b0_v, acc_sh.at[i0_v], add=True)
        pltpu.sync_copy(b1_v, acc_sh.at[i1_v], add=True)
        pltpu.sync_copy(b2_v, acc_sh.at[i2_v], add=True)

    plsc.subcore_barrier()

    pltpu.sync_copy(acc_sh.at[pl.ds(s * ACC_PER_SUB, ACC_PER_SUB)], w_v)
    pltpu.sync_copy(w_v, out_hbm.at[c, pl.ds(s * ACC_PER_SUB, ACC_PER_SUB)])


@functools.cache
def _scatter_call():
    mesh = plsc.VectorSubcoreMesh(
        core_axis_name="c", subcore_axis_name="s",
        num_cores=NCORES, num_subcores=NSUB,
    )
    return pl.kernel(
        _scatter_body,
        mesh=mesh,
        out_type=jax.ShapeDtypeStruct((NCORES, ACC_R, C_CH), jnp.float32),
        compiler_params=pltpu.CompilerParams(use_tc_tiling_on_sc=False),
        scratch_types=[
            pltpu.VMEM((PER_TILE,), jnp.int32),
            pltpu.VMEM((PER_TILE,), jnp.int32),
            pltpu.VMEM((PER_TILE,), jnp.int32),
            pltpu.VMEM((PER_TILE, C_CH), jnp.float32),
            pltpu.VMEM((PER_TILE, C_CH), jnp.float32),
            pltpu.VMEM((PER_TILE, C_CH), jnp.float32),
            pltpu.VMEM((ACC_PER_SUB, C_CH), jnp.float32),
            pltpu.VMEM_SHARED((ACC_R, C_CH), jnp.float32),
        ],
    )


def _finish_body(parts_ref, aux_ref, o_ref):
    acc = parts_ref[0] + parts_ref[1]
    st = acc[2 * NCELL:]
    base = float(NSRC) - st[:, 2:3]
    aux = aux_ref[...]
    for n in range(N_B):
        denom = st[:, n:n + 1] + base * aux[n:n + 1]
        numer = acc[n * NCELL:(n + 1) * NCELL]
        o_ref[n] = (numer / denom).T


def _finish_call(parts, aux):
    return pl.pallas_call(
        _finish_body,
        out_shape=jax.ShapeDtypeStruct((N_B, C_CH, NCELL), jnp.float32),
    )(parts, aux)


def kernel(X, field, convert):
    rows0, rows1, stats, dest3, aux = _prep_call(X, field, convert.reshape(N_B, 1))
    parts = _scatter_call()(rows0, rows1, stats, dest3.reshape(NBAND * NSRC))
    out = _finish_call(parts, aux)
    return out.reshape(N_B, C_CH, OUT, OUT)
